# Initial kernel scaffold; baseline (speedup 1.0000x reference)
import jax
import jax.numpy as jnp
from jax.experimental import pallas as pl


def kernel(x, weight):
    raise NotImplementedError("write your pallas kernel here")


# speedup vs baseline: 1.1013x; 1.1013x over previous
import functools

import jax
import jax.numpy as jnp
from jax import lax
from jax.experimental import pallas as pl
from jax.experimental.pallas import tpu as pltpu
from jax.experimental.pallas import tpu_sc as plsc

_NC = 2
_NS = 16
_NW = _NC * _NS
_CH = 128


def _gather_body(table_hbm, idx_hbm, out_hbm, idx_v, rows_v, gsem):
    wid = lax.axis_index("s") * _NC + lax.axis_index("c")
    nchunk = idx_v.shape[0]
    pltpu.sync_copy(idx_hbm.at[wid], idx_v)

    def step(j, carry):
        pltpu.async_copy(table_hbm.at[idx_v.at[j]], rows_v, gsem).wait()
        pltpu.sync_copy(rows_v, out_hbm.at[wid, j])
        return carry

    lax.fori_loop(0, nchunk, step, 0)


def kernel(x, weight):
    batch, fields = x.shape
    depth = weight.shape[1]
    total = batch * fields
    nchunk = total // (_NW * _CH)
    idx = x.reshape(_NW, nchunk, _CH)

    call = pl.kernel(
        _gather_body,
        out_type=jax.ShapeDtypeStruct((_NW, nchunk, _CH, depth), jnp.float32),
        mesh=plsc.VectorSubcoreMesh(core_axis_name="c", subcore_axis_name="s"),
        scratch_types=[
            pltpu.VMEM((nchunk, _CH), jnp.int32),
            pltpu.VMEM((_CH, depth), jnp.float32),
            pltpu.SemaphoreType.DMA,
        ],
        compiler_params=pltpu.CompilerParams(use_tc_tiling_on_sc=False),
    )
    out = call(weight, idx)
    return out.reshape(batch, fields, depth)


# speedup vs baseline: 1.1887x; 1.0794x over previous
import functools

import jax
import jax.numpy as jnp
from jax import lax
from jax.experimental import pallas as pl
from jax.experimental.pallas import tpu as pltpu
from jax.experimental.pallas import tpu_sc as plsc

_NC = 2
_NS = 16
_NW = _NC * _NS
_CH = 128


_NBUF = 2


def _gather_body(table_hbm, idx_hbm, out_hbm, idx_v, rows_v, *sems):
    gsems, osems = sems[:_NBUF], sems[_NBUF:]
    wid = lax.axis_index("s") * _NC + lax.axis_index("c")
    nchunk = idx_v.shape[0]
    pltpu.sync_copy(idx_hbm.at[wid], idx_v)

    pltpu.async_copy(table_hbm.at[idx_v.at[0]], rows_v.at[0], gsems[0])

    @pl.loop(0, nchunk, step=_NBUF)
    def _outer(g):
        for b in range(_NBUF):
            j = g + b
            slot = b
            nxt = (b + 1) % _NBUF

            @pl.when(j >= 1)
            def _():
                pltpu.make_async_copy(
                    rows_v.at[nxt], out_hbm.at[wid, 0], osems[nxt]
                ).wait()

            @pl.when(j + 1 < nchunk)
            def _():
                pltpu.async_copy(
                    table_hbm.at[idx_v.at[j + 1]], rows_v.at[nxt], gsems[nxt]
                )

            pltpu.make_async_copy(
                table_hbm.at[idx_v.at[0]], rows_v.at[slot], gsems[slot]
            ).wait()
            pltpu.async_copy(rows_v.at[slot], out_hbm.at[wid, j], osems[slot])

    last = (nchunk - 1) % _NBUF
    pltpu.make_async_copy(rows_v.at[last], out_hbm.at[wid, 0], osems[last]).wait()


def kernel(x, weight):
    batch, fields = x.shape
    depth = weight.shape[1]
    total = batch * fields
    nchunk = total // (_NW * _CH)
    idx = x.reshape(_NW, nchunk, _CH)

    call = pl.kernel(
        _gather_body,
        out_type=jax.ShapeDtypeStruct((_NW, nchunk, _CH, depth), jnp.float32),
        mesh=plsc.VectorSubcoreMesh(core_axis_name="c", subcore_axis_name="s"),
        scratch_types=[
            pltpu.VMEM((nchunk, _CH), jnp.int32),
            pltpu.VMEM((_NBUF, _CH, depth), jnp.float32),
        ] + [pltpu.SemaphoreType.DMA] * (2 * _NBUF),
        compiler_params=pltpu.CompilerParams(use_tc_tiling_on_sc=False),
    )
    out = call(weight, idx)
    return out.reshape(batch, fields, depth)
